# baseline (device time: 20925 ns/iter reference)
import jax
import jax.numpy as jnp
from jax import lax
from jax.experimental import pallas as pl
from jax.experimental.pallas import tpu as pltpu

NCH = 16
NBLK = 4


def kernel(x, dy):
    k, d = x.shape
    _, f = dy.shape
    half = d // 2
    cw = f // NCH
    bw = f // NBLK
    cpb = NCH // NBLK

    def body(x_hbm, dy_hbm, out_hbm,
             x_ref, dy_ref, pmine_buf, psend_buf, yrecv_buf, sred_buf,
             in_sems, out_sems, ysend_sems, yrecv_sems):
        my_x = lax.axis_index("x")
        my_y = lax.axis_index("y")
        my_z = lax.axis_index("z")
        ypartner = (my_x, 1 - my_y, my_z)

        x_copy = pltpu.make_async_copy(x_hbm, x_ref, in_sems.at[0])
        dy_copy = pltpu.make_async_copy(dy_hbm, dy_ref, in_sems.at[1])
        x_copy.start()
        dy_copy.start()

        barrier = pltpu.get_barrier_semaphore()
        pl.semaphore_signal(
            barrier, inc=1, device_id=ypartner,
            device_id_type=pl.DeviceIdType.MESH,
        )
        pl.semaphore_wait(barrier, 1)
        x_copy.wait()
        dy_copy.wait()

        def y_rdma(j):
            return pltpu.make_async_remote_copy(
                src_ref=psend_buf.at[j],
                dst_ref=yrecv_buf.at[j],
                send_sem=ysend_sems.at[j],
                recv_sem=yrecv_sems.at[j],
                device_id=ypartner,
                device_id_type=pl.DeviceIdType.MESH,
            )

        def impl(mine, theirs):
            for b in range(NBLK):
                p = lax.dot_general(
                    x_ref[...], dy_ref[:, b * bw:(b + 1) * bw],
                    dimension_numbers=(((0,), (0,)), ((), ())),
                    preferred_element_type=jnp.float32,
                )
                for i in range(cpb):
                    j = b * cpb + i
                    sl = p[:, i * cw:(i + 1) * cw]
                    pmine_buf[j] = sl[mine:mine + half]
                    psend_buf[j] = sl[theirs:theirs + half].astype(jnp.bfloat16)
                    y_rdma(j).start()

            for j in range(NCH):
                y_rdma(j).wait_recv()
                sred_buf[j] = pmine_buf[j] + yrecv_buf[j].astype(jnp.float32)
                pltpu.make_async_copy(
                    sred_buf.at[j],
                    out_hbm.at[:, j * cw:(j + 1) * cw],
                    out_sems.at[j],
                ).start()

            for j in range(NCH):
                y_rdma(j).wait_send()
                pltpu.make_async_copy(
                    sred_buf.at[j],
                    out_hbm.at[:, j * cw:(j + 1) * cw],
                    out_sems.at[j],
                ).wait()

        pl.when(my_y == 0)(lambda: impl(0, half))
        pl.when(my_y == 1)(lambda: impl(half, 0))

    return pl.pallas_call(
        body,
        out_shape=jax.ShapeDtypeStruct((half, f), jnp.float32),
        in_specs=[
            pl.BlockSpec(memory_space=pl.ANY),
            pl.BlockSpec(memory_space=pl.ANY),
        ],
        out_specs=pl.BlockSpec(memory_space=pl.ANY),
        scratch_shapes=[
            pltpu.VMEM((k, d), jnp.float32),
            pltpu.VMEM((k, f), jnp.float32),
            pltpu.VMEM((NCH, half, cw), jnp.float32),
            pltpu.VMEM((NCH, half, cw), jnp.bfloat16),
            pltpu.VMEM((NCH, half, cw), jnp.bfloat16),
            pltpu.VMEM((NCH, half, cw), jnp.float32),
            pltpu.SemaphoreType.DMA((2,)),
            pltpu.SemaphoreType.DMA((NCH,)),
            pltpu.SemaphoreType.DMA((NCH,)),
            pltpu.SemaphoreType.DMA((NCH,)),
        ],
        compiler_params=pltpu.CompilerParams(collective_id=0),
    )(x, dy)


# device time: 20205 ns/iter; 1.0356x vs baseline; 1.0356x over previous
import jax
import jax.numpy as jnp
from jax import lax
from jax.experimental import pallas as pl
from jax.experimental.pallas import tpu as pltpu

NCH = 16
BLOCK_WIDTHS = (256, 256, 512, 512, 512)


def kernel(x, dy):
    k, d = x.shape
    _, f = dy.shape
    half = d // 2
    cw = f // NCH

    def body(x_ref, dy_ref, out_ref,
             pmine_buf, psend_buf, yrecv_buf,
             ysend_sems, yrecv_sems):
        my_x = lax.axis_index("x")
        my_y = lax.axis_index("y")
        my_z = lax.axis_index("z")
        ypartner = (my_x, 1 - my_y, my_z)

        barrier = pltpu.get_barrier_semaphore()
        pl.semaphore_signal(
            barrier, inc=1, device_id=ypartner,
            device_id_type=pl.DeviceIdType.MESH,
        )
        pl.semaphore_wait(barrier, 1)

        def y_rdma(j):
            return pltpu.make_async_remote_copy(
                src_ref=psend_buf.at[j],
                dst_ref=yrecv_buf.at[j],
                send_sem=ysend_sems.at[j],
                recv_sem=yrecv_sems.at[j],
                device_id=ypartner,
                device_id_type=pl.DeviceIdType.MESH,
            )

        def impl(mine, theirs):
            col = 0
            for bw in BLOCK_WIDTHS:
                p = lax.dot_general(
                    x_ref[...], dy_ref[:, col:col + bw],
                    dimension_numbers=(((0,), (0,)), ((), ())),
                    preferred_element_type=jnp.float32,
                )
                for i in range(bw // cw):
                    j = col // cw + i
                    sl = p[:, i * cw:(i + 1) * cw]
                    pmine_buf[j] = sl[mine:mine + half]
                    psend_buf[j] = sl[theirs:theirs + half].astype(jnp.bfloat16)
                    y_rdma(j).start()
                col += bw

            for j in range(NCH):
                y_rdma(j).wait_recv()
                out_ref[:, j * cw:(j + 1) * cw] = (
                    pmine_buf[j] + yrecv_buf[j].astype(jnp.float32)
                )

            for j in range(NCH):
                y_rdma(j).wait_send()

        pl.when(my_y == 0)(lambda: impl(0, half))
        pl.when(my_y == 1)(lambda: impl(half, 0))

    return pl.pallas_call(
        body,
        out_shape=jax.ShapeDtypeStruct((half, f), jnp.float32),
        in_specs=[
            pl.BlockSpec(memory_space=pltpu.VMEM),
            pl.BlockSpec(memory_space=pltpu.VMEM),
        ],
        out_specs=pl.BlockSpec(memory_space=pltpu.VMEM),
        scratch_shapes=[
            pltpu.VMEM((NCH, half, cw), jnp.float32),
            pltpu.VMEM((NCH, half, cw), jnp.bfloat16),
            pltpu.VMEM((NCH, half, cw), jnp.bfloat16),
            pltpu.SemaphoreType.DMA((NCH,)),
            pltpu.SemaphoreType.DMA((NCH,)),
        ],
        compiler_params=pltpu.CompilerParams(collective_id=0),
    )(x, dy)


# device time: 18533 ns/iter; 1.1291x vs baseline; 1.0902x over previous
import jax
import jax.numpy as jnp
from jax import lax
from jax.experimental import pallas as pl
from jax.experimental.pallas import tpu as pltpu

NCH = 8
BLOCK_WIDTHS = (256, 256, 512)


def kernel(x, dy):
    k, d = x.shape
    _, f = dy.shape
    half = d // 2
    fx = f // 2
    cw = fx // NCH

    def body(x_ref, dy_ref, out_ref,
             dyblk, pmine_buf, psend_buf, yrecv_buf,
             sredf_buf, sred16_buf, xrecv_buf, convf_buf,
             dy_sem, outm_sems, outo_sems,
             ysend_sems, yrecv_sems, xsend_sems, xrecv_sems):
        my_x = lax.axis_index("x")
        my_y = lax.axis_index("y")
        my_z = lax.axis_index("z")
        ypartner = (my_x, 1 - my_y, my_z)
        xpartner = (1 - my_x, my_y, my_z)

        col0 = my_x * fx
        other0 = (1 - my_x) * fx

        dy_copy = pltpu.make_async_copy(
            dy_ref.at[:, pl.ds(col0, fx)], dyblk, dy_sem
        )
        dy_copy.start()

        barrier = pltpu.get_barrier_semaphore()
        for nbr in (ypartner, xpartner):
            pl.semaphore_signal(
                barrier, inc=1, device_id=nbr,
                device_id_type=pl.DeviceIdType.MESH,
            )
        pl.semaphore_wait(barrier, 2)
        dy_copy.wait()

        def y_rdma(j):
            return pltpu.make_async_remote_copy(
                src_ref=psend_buf.at[j],
                dst_ref=yrecv_buf.at[j],
                send_sem=ysend_sems.at[j],
                recv_sem=yrecv_sems.at[j],
                device_id=ypartner,
                device_id_type=pl.DeviceIdType.MESH,
            )

        def x_rdma(j):
            return pltpu.make_async_remote_copy(
                src_ref=sred16_buf.at[j],
                dst_ref=xrecv_buf.at[j],
                send_sem=xsend_sems.at[j],
                recv_sem=xrecv_sems.at[j],
                device_id=xpartner,
                device_id_type=pl.DeviceIdType.MESH,
            )

        def impl(mine, theirs):
            col = 0
            for bw in BLOCK_WIDTHS:
                p = lax.dot_general(
                    x_ref[...], dyblk[:, col:col + bw],
                    dimension_numbers=(((0,), (0,)), ((), ())),
                    preferred_element_type=jnp.float32,
                )
                for i in range(bw // cw):
                    j = col // cw + i
                    sl = p[:, i * cw:(i + 1) * cw]
                    pmine_buf[j] = sl[mine:mine + half]
                    psend_buf[j] = sl[theirs:theirs + half].astype(jnp.bfloat16)
                    y_rdma(j).start()
                col += bw

            for j in range(NCH):
                y_rdma(j).wait_recv()
                s = pmine_buf[j] + yrecv_buf[j].astype(jnp.float32)
                sredf_buf[j] = s
                sred16_buf[j] = s.astype(jnp.bfloat16)
                x_rdma(j).start()
                pltpu.make_async_copy(
                    sredf_buf.at[j],
                    out_ref.at[:, pl.ds(col0 + j * cw, cw)],
                    outm_sems.at[j],
                ).start()

            for j in range(NCH):
                x_rdma(j).wait_recv()
                convf_buf[j] = xrecv_buf[j].astype(jnp.float32)
                pltpu.make_async_copy(
                    convf_buf.at[j],
                    out_ref.at[:, pl.ds(other0 + j * cw, cw)],
                    outo_sems.at[j],
                ).start()

            for j in range(NCH):
                pltpu.make_async_copy(
                    sredf_buf.at[j],
                    out_ref.at[:, pl.ds(col0 + j * cw, cw)],
                    outm_sems.at[j],
                ).wait()
                pltpu.make_async_copy(
                    convf_buf.at[j],
                    out_ref.at[:, pl.ds(other0 + j * cw, cw)],
                    outo_sems.at[j],
                ).wait()
                y_rdma(j).wait_send()
                x_rdma(j).wait_send()

        pl.when(my_y == 0)(lambda: impl(0, half))
        pl.when(my_y == 1)(lambda: impl(half, 0))

    return pl.pallas_call(
        body,
        out_shape=jax.ShapeDtypeStruct((half, f), jnp.float32),
        in_specs=[
            pl.BlockSpec(memory_space=pltpu.VMEM),
            pl.BlockSpec(memory_space=pltpu.VMEM),
        ],
        out_specs=pl.BlockSpec(memory_space=pltpu.VMEM),
        scratch_shapes=[
            pltpu.VMEM((k, fx), jnp.float32),
            pltpu.VMEM((NCH, half, cw), jnp.float32),
            pltpu.VMEM((NCH, half, cw), jnp.bfloat16),
            pltpu.VMEM((NCH, half, cw), jnp.bfloat16),
            pltpu.VMEM((NCH, half, cw), jnp.float32),
            pltpu.VMEM((NCH, half, cw), jnp.bfloat16),
            pltpu.VMEM((NCH, half, cw), jnp.bfloat16),
            pltpu.VMEM((NCH, half, cw), jnp.float32),
            pltpu.SemaphoreType.DMA,
            pltpu.SemaphoreType.DMA((NCH,)),
            pltpu.SemaphoreType.DMA((NCH,)),
            pltpu.SemaphoreType.DMA((NCH,)),
            pltpu.SemaphoreType.DMA((NCH,)),
            pltpu.SemaphoreType.DMA((NCH,)),
            pltpu.SemaphoreType.DMA((NCH,)),
        ],
        compiler_params=pltpu.CompilerParams(collective_id=0),
    )(x, dy)


# device time: 18094 ns/iter; 1.1565x vs baseline; 1.0243x over previous
import jax
import jax.numpy as jnp
from jax import lax
from jax.experimental import pallas as pl
from jax.experimental.pallas import tpu as pltpu

NCH = 4
X_FIRST = (2, 3)
Z_FIRST = (0, 1)
Y_ORDER = (0, 2, 1, 3)


def kernel(x, dy):
    k, d = x.shape
    _, f = dy.shape
    half = d // 2
    fq = f // 4
    cw = fq // NCH

    def body(x_ref, dy_ref, out_ref,
             dyblk, pmine_buf, psend_buf, yrecv_buf,
             sredf_buf, sred16_buf,
             xrecv_m, zrecv_m, xrecv_d, zrecv_d,
             convx, convz, convd,
             dy_sem, out_sems,
             ys_sems, yr_sems, xs_sems, xr_sems, zs_sems, zr_sems,
             fxs_sems, fxr_sems, fzs_sems, fzr_sems):
        my_x = lax.axis_index("x")
        my_y = lax.axis_index("y")
        my_z = lax.axis_index("z")
        zp = my_z % 2
        zpartner_z = my_z - zp + (1 - zp)
        ypartner = (my_x, 1 - my_y, my_z)
        xpartner = (1 - my_x, my_y, my_z)
        zpartner = (my_x, my_y, zpartner_z)

        cq_me = (2 * my_x + zp) * fq
        cq_x = (2 * (1 - my_x) + zp) * fq
        cq_z = (2 * my_x + (1 - zp)) * fq
        cq_d = (2 * (1 - my_x) + (1 - zp)) * fq

        dy_copy = pltpu.make_async_copy(
            dy_ref.at[:, pl.ds(cq_me, fq)], dyblk, dy_sem
        )
        dy_copy.start()

        barrier = pltpu.get_barrier_semaphore()
        for nbr in (ypartner, xpartner, zpartner):
            pl.semaphore_signal(
                barrier, inc=1, device_id=nbr,
                device_id_type=pl.DeviceIdType.MESH,
            )
        pl.semaphore_wait(barrier, 3)
        dy_copy.wait()

        def y_rdma(j):
            return pltpu.make_async_remote_copy(
                src_ref=psend_buf.at[j], dst_ref=yrecv_buf.at[j],
                send_sem=ys_sems.at[j], recv_sem=yr_sems.at[j],
                device_id=ypartner, device_id_type=pl.DeviceIdType.MESH,
            )

        def xm_rdma(j):
            return pltpu.make_async_remote_copy(
                src_ref=sred16_buf.at[j], dst_ref=xrecv_m.at[j],
                send_sem=xs_sems.at[j], recv_sem=xr_sems.at[j],
                device_id=xpartner, device_id_type=pl.DeviceIdType.MESH,
            )

        def zm_rdma(j):
            return pltpu.make_async_remote_copy(
                src_ref=sred16_buf.at[j], dst_ref=zrecv_m.at[j],
                send_sem=zs_sems.at[j], recv_sem=zr_sems.at[j],
                device_id=zpartner, device_id_type=pl.DeviceIdType.MESH,
            )

        def fz_rdma(i):
            return pltpu.make_async_remote_copy(
                src_ref=xrecv_m.at[X_FIRST[i]], dst_ref=zrecv_d.at[i],
                send_sem=fzs_sems.at[i], recv_sem=fzr_sems.at[i],
                device_id=zpartner, device_id_type=pl.DeviceIdType.MESH,
            )

        def fx_rdma(i):
            return pltpu.make_async_remote_copy(
                src_ref=zrecv_m.at[Z_FIRST[i]], dst_ref=xrecv_d.at[i],
                send_sem=fxs_sems.at[i], recv_sem=fxr_sems.at[i],
                device_id=xpartner, device_id_type=pl.DeviceIdType.MESH,
            )

        def store(buf_slot, col, sem):
            return pltpu.make_async_copy(
                buf_slot, out_ref.at[:, pl.ds(col, cw)], sem
            )

        def impl(mine, theirs):
            for j in Y_ORDER:
                p = lax.dot_general(
                    x_ref[...], dyblk[:, j * cw:(j + 1) * cw],
                    dimension_numbers=(((0,), (0,)), ((), ())),
                    preferred_element_type=jnp.float32,
                )
                pmine_buf[j] = p[mine:mine + half]
                psend_buf[j] = p[theirs:theirs + half].astype(jnp.bfloat16)
                y_rdma(j).start()

            for j in Y_ORDER:
                y_rdma(j).wait_recv()
                s = pmine_buf[j] + yrecv_buf[j].astype(jnp.float32)
                sredf_buf[j] = s
                sred16_buf[j] = s.astype(jnp.bfloat16)
                if j in X_FIRST:
                    xm_rdma(j).start()
                else:
                    zm_rdma(j).start()
                store(sredf_buf.at[j], cq_me + j * cw, out_sems.at[j]).start()
            for j in Z_FIRST:
                xm_rdma(j).start()
            for j in X_FIRST:
                zm_rdma(j).start()

            for i in range(2):
                xm_rdma(X_FIRST[i]).wait_recv()
                fz_rdma(i).start()
            for i in range(2):
                zm_rdma(Z_FIRST[i]).wait_recv()
                fx_rdma(i).start()

            for j in Z_FIRST:
                xm_rdma(j).wait_recv()
            for j in X_FIRST:
                zm_rdma(j).wait_recv()
            for j in range(NCH):
                convx[j] = xrecv_m[j].astype(jnp.float32)
                store(convx.at[j], cq_x + j * cw, out_sems.at[4 + j]).start()
                convz[j] = zrecv_m[j].astype(jnp.float32)
                store(convz.at[j], cq_z + j * cw, out_sems.at[8 + j]).start()

            for i in range(2):
                fx_rdma(i).wait_recv()
                convd[Z_FIRST[i]] = xrecv_d[i].astype(jnp.float32)
                store(convd.at[Z_FIRST[i]], cq_d + Z_FIRST[i] * cw,
                      out_sems.at[12 + Z_FIRST[i]]).start()
            for i in range(2):
                fz_rdma(i).wait_recv()
                convd[X_FIRST[i]] = zrecv_d[i].astype(jnp.float32)
                store(convd.at[X_FIRST[i]], cq_d + X_FIRST[i] * cw,
                      out_sems.at[12 + X_FIRST[i]]).start()

            for j in range(NCH):
                y_rdma(j).wait_send()
                xm_rdma(j).wait_send()
                zm_rdma(j).wait_send()
            for i in range(2):
                fx_rdma(i).wait_send()
                fz_rdma(i).wait_send()
            for j in range(NCH):
                store(sredf_buf.at[j], cq_me + j * cw, out_sems.at[j]).wait()
                store(convx.at[j], cq_x + j * cw, out_sems.at[4 + j]).wait()
                store(convz.at[j], cq_z + j * cw, out_sems.at[8 + j]).wait()
                store(convd.at[j], cq_d + j * cw, out_sems.at[12 + j]).wait()

        pl.when(my_y == 0)(lambda: impl(0, half))
        pl.when(my_y == 1)(lambda: impl(half, 0))

    return pl.pallas_call(
        body,
        out_shape=jax.ShapeDtypeStruct((half, f), jnp.float32),
        in_specs=[
            pl.BlockSpec(memory_space=pltpu.VMEM),
            pl.BlockSpec(memory_space=pltpu.VMEM),
        ],
        out_specs=pl.BlockSpec(memory_space=pltpu.VMEM),
        scratch_shapes=[
            pltpu.VMEM((k, fq), jnp.float32),
            pltpu.VMEM((NCH, half, cw), jnp.float32),
            pltpu.VMEM((NCH, half, cw), jnp.bfloat16),
            pltpu.VMEM((NCH, half, cw), jnp.bfloat16),
            pltpu.VMEM((NCH, half, cw), jnp.float32),
            pltpu.VMEM((NCH, half, cw), jnp.bfloat16),
            pltpu.VMEM((NCH, half, cw), jnp.bfloat16),
            pltpu.VMEM((NCH, half, cw), jnp.bfloat16),
            pltpu.VMEM((2, half, cw), jnp.bfloat16),
            pltpu.VMEM((2, half, cw), jnp.bfloat16),
            pltpu.VMEM((NCH, half, cw), jnp.float32),
            pltpu.VMEM((NCH, half, cw), jnp.float32),
            pltpu.VMEM((NCH, half, cw), jnp.float32),
            pltpu.SemaphoreType.DMA,
            pltpu.SemaphoreType.DMA((16,)),
            pltpu.SemaphoreType.DMA((NCH,)),
            pltpu.SemaphoreType.DMA((NCH,)),
            pltpu.SemaphoreType.DMA((NCH,)),
            pltpu.SemaphoreType.DMA((NCH,)),
            pltpu.SemaphoreType.DMA((NCH,)),
            pltpu.SemaphoreType.DMA((NCH,)),
            pltpu.SemaphoreType.DMA((2,)),
            pltpu.SemaphoreType.DMA((2,)),
            pltpu.SemaphoreType.DMA((2,)),
            pltpu.SemaphoreType.DMA((2,)),
        ],
        compiler_params=pltpu.CompilerParams(collective_id=0),
    )(x, dy)
